# baseline (speedup 1.0000x reference)
import functools

import jax
import jax.numpy as jnp
from jax import lax
from jax.experimental import pallas as pl
from jax.experimental.pallas import tpu as pltpu
from jax.experimental.pallas import tpu_sc as plsc

N_NODES = 100000
EMBED_DIM = 128

_info = plsc.get_sparse_core_info()
_NC, _NS = _info.num_cores, _info.num_subcores
_NW = _NC * _NS
_ELEMS = N_NODES * EMBED_DIM
_ELEMS_PER_W = _ELEMS // _NW
_BUF_ROWS = 625
_BUF = _BUF_ROWS * EMBED_DIM
_SMALL_ROWS = 125
_SMALL = _SMALL_ROWS * EMBED_DIM
_N_SP = 12
_SP_PART = _N_SP * _SMALL
_TS_PART = _ELEMS_PER_W - _SP_PART
_LANES = 16
_FILL_UNROLL = 4

_mesh = plsc.VectorSubcoreMesh(core_axis_name="c", subcore_axis_name="s")


@functools.partial(
    pl.kernel,
    mesh=_mesh,
    out_type=jax.ShapeDtypeStruct((_ELEMS,), jnp.float32),
    scratch_types=[
        pltpu.VMEM((_BUF,), jnp.float32),
        pltpu.VMEM_SHARED((_NS * _SMALL,), jnp.float32),
        pltpu.SemaphoreType.DMA,
    ],
)
def _broadcast_rows(w_hbm, out_hbm, buf_v, sh_v, sem):
    sid = lax.axis_index("s")
    wid = sid * _NC + lax.axis_index("c")
    base = wid * _ELEMS_PER_W
    pltpu.sync_copy(w_hbm, buf_v.at[pl.ds(0, EMBED_DIM)])
    wv = [buf_v[pl.ds(d * _LANES, _LANES)] for d in range(EMBED_DIM // _LANES)]

    def _fill_rows(first_row):
        def body(i, _):
            for u in range(_FILL_UNROLL):
                row = (first_row + u) * EMBED_DIM + i * (_FILL_UNROLL * EMBED_DIM)
                for d in range(EMBED_DIM // _LANES):
                    buf_v[pl.ds(row + d * _LANES, _LANES)] = wv[d]
            return 0
        return body

    lax.fori_loop(0, (_SMALL_ROWS - 1) // _FILL_UNROLL, _fill_rows(1), 0)
    slot = sh_v.at[pl.ds(sid * _SMALL, _SMALL)]
    pltpu.sync_copy(buf_v.at[pl.ds(0, _SMALL)], slot)
    copies = [
        pltpu.async_copy(
            slot, out_hbm.at[pl.ds(base + _TS_PART + j * _SMALL, _SMALL)], sem)
        for j in range(_N_SP)
    ]
    copies += [
        pltpu.async_copy(
            buf_v.at[pl.ds(0, _SMALL)],
            out_hbm.at[pl.ds(base + j * _SMALL, _SMALL)], sem)
        for j in range(3)
    ]
    lax.fori_loop(0, (_BUF_ROWS - _SMALL_ROWS) // _FILL_UNROLL,
                  _fill_rows(_SMALL_ROWS), 0)
    copies += [
        pltpu.async_copy(
            buf_v, out_hbm.at[pl.ds(base + 3 * _SMALL + j * _BUF, _BUF)], sem)
        for j in range(2)
    ]
    for c in copies:
        c.wait()


def kernel(tensor, weight):
    del tensor
    flat = _broadcast_rows(weight.reshape(EMBED_DIM))
    return flat.reshape(N_NODES, EMBED_DIM)


# speedup vs baseline: 1.0354x; 1.0354x over previous
import functools

import jax
import jax.numpy as jnp
from jax import lax
from jax.experimental import pallas as pl
from jax.experimental.pallas import tpu as pltpu
from jax.experimental.pallas import tpu_sc as plsc

N_NODES = 100000
EMBED_DIM = 128

_info = plsc.get_sparse_core_info()
_NC, _NS = _info.num_cores, _info.num_subcores
_NW = _NC * _NS
_ELEMS = N_NODES * EMBED_DIM
_ELEMS_PER_W = _ELEMS // _NW
_BUF_ROWS = 625
_BUF = _BUF_ROWS * EMBED_DIM
_SMALL_ROWS = 125
_SMALL = _SMALL_ROWS * EMBED_DIM
_N_SMALL = 5
_N_BIG = 4
_LANES = 16
_FILL_UNROLL = 4

_mesh = plsc.VectorSubcoreMesh(core_axis_name="c", subcore_axis_name="s")


@functools.partial(
    pl.kernel,
    mesh=_mesh,
    out_type=jax.ShapeDtypeStruct((_ELEMS,), jnp.float32),
    scratch_types=[
        pltpu.VMEM((_BUF,), jnp.float32),
        pltpu.SemaphoreType.DMA,
    ],
)
def _broadcast_rows(w_hbm, out_hbm, buf_v, sem):
    wid = lax.axis_index("s") * _NC + lax.axis_index("c")
    pltpu.sync_copy(w_hbm, buf_v.at[pl.ds(0, EMBED_DIM)])
    wv = [buf_v[pl.ds(d * _LANES, _LANES)] for d in range(EMBED_DIM // _LANES)]

    def _fill_rows(first_row):
        def body(i, _):
            for u in range(_FILL_UNROLL):
                row = (first_row + u) * EMBED_DIM + i * (_FILL_UNROLL * EMBED_DIM)
                for d in range(EMBED_DIM // _LANES):
                    buf_v[pl.ds(row + d * _LANES, _LANES)] = wv[d]
            return 0
        return body

    base = wid * _ELEMS_PER_W
    lax.fori_loop(0, (_SMALL_ROWS - 1) // _FILL_UNROLL, _fill_rows(1), 0)
    copies = [
        pltpu.async_copy(
            buf_v.at[pl.ds(0, _SMALL)],
            out_hbm.at[pl.ds(base + j * _SMALL, _SMALL)], sem)
        for j in range(_N_SMALL)
    ]
    lax.fori_loop(0, (_BUF_ROWS - _SMALL_ROWS) // _FILL_UNROLL,
                  _fill_rows(_SMALL_ROWS), 0)
    off = _N_SMALL * _SMALL
    copies += [
        pltpu.async_copy(buf_v, out_hbm.at[pl.ds(base + off + j * _BUF, _BUF)], sem)
        for j in range(_N_BIG)
    ]
    for c in copies:
        c.wait()


def kernel(tensor, weight):
    del tensor
    flat = _broadcast_rows(weight.reshape(EMBED_DIM))
    return flat.reshape(N_NODES, EMBED_DIM)
